# baseline (speedup 1.0000x reference)
import functools

import jax
import jax.numpy as jnp
from jax import lax
from jax.experimental import pallas as pl
from jax.experimental.pallas import tpu as pltpu
from jax.experimental.pallas import tpu_sc as plsc

B, C, N, T = 8, 64, 10000, 12
BT = B * T
TIME_V = 288 + 1
DAY_V = 7 + 1
DEG_V = 64 + 1

NW = 32
RPW = 320
R = NW * RPW
CW = 128
CHUNK = 128
NBLK = (N + CHUNK - 1) // CHUNK


def _sc_gather_body(nid_ref, ntab_ref, nrows_ref, nidx, nbuf, sem):
    wid = lax.axis_index("s") * 2 + lax.axis_index("c")
    base = wid * RPW
    pltpu.sync_copy(nid_ref.at[wid], nidx)
    copies = []
    for lo in range(0, RPW, CHUNK):
        sz = min(CHUNK, RPW - lo)
        copies.append(pltpu.async_copy(
            ntab_ref.at[nidx.at[pl.ds(lo, sz)]],
            nbuf.at[pl.ds(lo, sz)], sem))
    for cp in copies:
        cp.wait()
    pltpu.sync_copy(nbuf, nrows_ref.at[pl.ds(base, RPW)])


@functools.cache
def _make_sc_gather():
    return pl.kernel(
        _sc_gather_body,
        out_type=jax.ShapeDtypeStruct((R, CW), jnp.float32),
        mesh=plsc.VectorSubcoreMesh(core_axis_name="c", subcore_axis_name="s"),
        scratch_types=[pltpu.VMEM((RPW,), jnp.int32),
                       pltpu.VMEM((RPW, CW), jnp.float32),
                       pltpu.SemaphoreType.DMA],
    )


def _prep_body(nr_ref, deg_ref, dtab_ref, eye_ref, iflat_ref, tt_ref, dt_ref,
               nadd_ref, bt_ref):
    nt = lax.dot_general(nr_ref[:, :C], eye_ref[...], (((0,), (0,)), ((), ())),
                         preferred_element_type=jnp.float32)
    d_ids = deg_ref[0]
    d_iota = lax.broadcasted_iota(jnp.int32, (DEG_V, CHUNK), 0)
    d_oh = (d_iota == d_ids).astype(jnp.float32)
    dt = lax.dot_general(dtab_ref[...], d_oh, (((0,), (0,)), ((), ())),
                         preferred_element_type=jnp.float32)
    nadd_ref[...] = nt + dt

    @pl.when(pl.program_id(0) == 0)
    def _bt():
        ids = iflat_ref[...]
        t_ids = ids[0:1, :]
        dy_ids = ids[1:2, :]
        t_iota = lax.broadcasted_iota(jnp.int32, (TIME_V, BT), 0)
        dy_iota = lax.broadcasted_iota(jnp.int32, (DAY_V, BT), 0)
        t_oh = (t_iota == t_ids).astype(jnp.float32)
        dy_oh = (dy_iota == dy_ids).astype(jnp.float32)
        bt = lax.dot_general(t_oh, tt_ref[...], (((0,), (0,)), ((), ())),
                             preferred_element_type=jnp.float32)
        bt = bt + lax.dot_general(dy_oh, dt_ref[...], (((0,), (0,)), ((), ())),
                                  preferred_element_type=jnp.float32)
        bt_ref[...] = bt[:, :, None]


def _prep(node_rows, deg3, dtab, eye, iflat, time_table, day_table):
    return pl.pallas_call(
        _prep_body,
        grid=(NBLK,),
        in_specs=[
            pl.BlockSpec((CHUNK, CW), lambda i: (i, 0)),
            pl.BlockSpec((1, 1, CHUNK), lambda i: (i, 0, 0)),
            pl.BlockSpec((DEG_V, C), lambda i: (0, 0)),
            pl.BlockSpec((CHUNK, CHUNK), lambda i: (0, 0)),
            pl.BlockSpec((2, BT), lambda i: (0, 0)),
            pl.BlockSpec((TIME_V, C), lambda i: (0, 0)),
            pl.BlockSpec((DAY_V, C), lambda i: (0, 0)),
        ],
        out_specs=[
            pl.BlockSpec((C, CHUNK), lambda i: (0, i)),
            pl.BlockSpec((BT, C, 1), lambda i: (0, 0, 0)),
        ],
        out_shape=[jax.ShapeDtypeStruct((C, N), jnp.float32),
                   jax.ShapeDtypeStruct((BT, C, 1), jnp.float32)],
    )(node_rows, deg3, dtab, eye, iflat, time_table, day_table)


RB = 4


def _main_body(x_ref, bt_ref, nadd_ref, out_ref):
    out_ref[...] = x_ref[...] + bt_ref[...] + nadd_ref[...][None, :, :]


def _main(y, btcol, nadd_t):
    return pl.pallas_call(
        _main_body,
        grid=(BT // RB,),
        in_specs=[
            pl.BlockSpec((RB, C, N), lambda i: (i, 0, 0)),
            pl.BlockSpec((RB, C, 1), lambda i: (i, 0, 0)),
            pl.BlockSpec((C, N), lambda i: (0, 0)),
        ],
        out_specs=pl.BlockSpec((RB, C, N), lambda i: (i, 0, 0)),
        out_shape=jax.ShapeDtypeStruct((BT, C, N), jnp.float32),
    )(y, btcol, nadd_t)


def kernel(x, idx, node_ids, degrees, time_table, day_table, node_table,
           degree_table):
    y = jnp.transpose(x, (0, 3, 1, 2)).reshape(BT, C, N)
    nid = jnp.concatenate(
        [node_ids, jnp.zeros((R - N,), jnp.int32)]).reshape(NW, RPW)
    ntab_p = jnp.pad(node_table, ((0, 0), (0, CW - C)))
    node_rows = _make_sc_gather()(nid, ntab_p)
    iflat = jnp.stack([idx[:, 0, :].reshape(BT), idx[:, 1, :].reshape(BT)])
    deg3 = jnp.concatenate(
        [degrees, jnp.zeros((NBLK * CHUNK - N,), jnp.int32)]
    ).reshape(NBLK, 1, CHUNK)
    eye = jnp.eye(CHUNK, dtype=jnp.float32)
    nadd_t, btcol = _prep(node_rows, deg3, degree_table, eye, iflat,
                          time_table, day_table)
    out = _main(y, btcol, nadd_t)
    return jnp.transpose(out.reshape(B, T, C, N), (0, 2, 3, 1))


# speedup vs baseline: 1.1303x; 1.1303x over previous
import functools

import jax
import jax.numpy as jnp
from jax import lax
from jax.experimental import pallas as pl
from jax.experimental.pallas import tpu as pltpu
from jax.experimental.pallas import tpu_sc as plsc

B, C, N, T = 8, 64, 10000, 12
BT = B * T
TIME_V = 288 + 1
DAY_V = 7 + 1
DEG_V = 64 + 1

NW = 32
RPW = 320
R = NW * RPW
CW = 128
CHUNK = 128
NBLK = (N + CHUNK - 1) // CHUNK


def _sc_gather_body(nid_ref, ntab_ref, nrows_ref, nidx, nbuf, sem):
    wid = lax.axis_index("s") * 2 + lax.axis_index("c")
    base = wid * RPW
    pltpu.sync_copy(nid_ref.at[wid], nidx)
    copies = []
    for lo in range(0, RPW, CHUNK):
        sz = min(CHUNK, RPW - lo)
        copies.append(pltpu.async_copy(
            ntab_ref.at[nidx.at[pl.ds(lo, sz)]],
            nbuf.at[pl.ds(lo, sz)], sem))
    for cp in copies:
        cp.wait()
    pltpu.sync_copy(nbuf, nrows_ref.at[pl.ds(base, RPW)])


@functools.cache
def _make_sc_gather():
    return pl.kernel(
        _sc_gather_body,
        out_type=jax.ShapeDtypeStruct((R, CW), jnp.float32),
        mesh=plsc.VectorSubcoreMesh(core_axis_name="c", subcore_axis_name="s"),
        scratch_types=[pltpu.VMEM((RPW,), jnp.int32),
                       pltpu.VMEM((RPW, CW), jnp.float32),
                       pltpu.SemaphoreType.DMA],
    )


def _bt_prep_body(iflat_ref, tt_ref, dt_ref, out_ref):
    ids = iflat_ref[...]
    t_ids = ids[0:1, :]
    d_ids = ids[1:2, :]
    t_iota = lax.broadcasted_iota(jnp.int32, (TIME_V, BT), 0)
    d_iota = lax.broadcasted_iota(jnp.int32, (DAY_V, BT), 0)
    t_oh = (t_iota == t_ids).astype(jnp.float32)
    d_oh = (d_iota == d_ids).astype(jnp.float32)
    bt = lax.dot_general(t_oh, tt_ref[...], (((0,), (0,)), ((), ())),
                         preferred_element_type=jnp.float32)
    bt = bt + lax.dot_general(d_oh, dt_ref[...], (((0,), (0,)), ((), ())),
                              preferred_element_type=jnp.float32)
    out_ref[...] = bt[:, :, None]


def _bt_prep(iflat, time_table, day_table):
    return pl.pallas_call(
        _bt_prep_body,
        grid=(1,),
        in_specs=[
            pl.BlockSpec((2, BT), lambda i: (0, 0)),
            pl.BlockSpec((TIME_V, C), lambda i: (0, 0)),
            pl.BlockSpec((DAY_V, C), lambda i: (0, 0)),
        ],
        out_specs=pl.BlockSpec((BT, C, 1), lambda i: (0, 0, 0)),
        out_shape=jax.ShapeDtypeStruct((BT, C, 1), jnp.float32),
    )(iflat, time_table, day_table)


RB = 4
NPAD = NBLK * CHUNK


def _main_body(nr_ref, deg_ref, dtab_ref, eye_ref, x_ref, bt_ref, out_ref,
               nadd):
    @pl.when(pl.program_id(0) == 0)
    def _init():
        def blk(k, _):
            off = pl.multiple_of(k * CHUNK, CHUNK)
            s = nr_ref[pl.ds(off, CHUNK), :C]
            nt = lax.dot_general(s, eye_ref[...], (((0,), (0,)), ((), ())),
                                 preferred_element_type=jnp.float32)
            d_ids = deg_ref[k]
            d_iota = lax.broadcasted_iota(jnp.int32, (DEG_V, CHUNK), 0)
            d_oh = (d_iota == d_ids).astype(jnp.float32)
            dt = lax.dot_general(dtab_ref[...], d_oh, (((0,), (0,)), ((), ())),
                                 preferred_element_type=jnp.float32)
            nadd[:, pl.ds(off, CHUNK)] = nt + dt
            return 0
        lax.fori_loop(0, NBLK, blk, 0)

    out_ref[...] = x_ref[...] + bt_ref[...] + nadd[:, :N][None, :, :]


def _main(node_rows, deg3, dtab, eye, y, btcol):
    return pl.pallas_call(
        _main_body,
        grid=(BT // RB,),
        in_specs=[
            pl.BlockSpec((R, CW), lambda i: (0, 0)),
            pl.BlockSpec((NBLK, 1, CHUNK), lambda i: (0, 0, 0)),
            pl.BlockSpec((DEG_V, C), lambda i: (0, 0)),
            pl.BlockSpec((CHUNK, CHUNK), lambda i: (0, 0)),
            pl.BlockSpec((RB, C, N), lambda i: (i, 0, 0)),
            pl.BlockSpec((RB, C, 1), lambda i: (i, 0, 0)),
        ],
        out_specs=pl.BlockSpec((RB, C, N), lambda i: (i, 0, 0)),
        out_shape=jax.ShapeDtypeStruct((BT, C, N), jnp.float32),
        scratch_shapes=[pltpu.VMEM((C, NPAD), jnp.float32)],
    )(node_rows, deg3, dtab, eye, y, btcol)


def kernel(x, idx, node_ids, degrees, time_table, day_table, node_table,
           degree_table):
    y = jnp.transpose(x, (0, 3, 1, 2)).reshape(BT, C, N)
    nid = jnp.concatenate(
        [node_ids, jnp.zeros((R - N,), jnp.int32)]).reshape(NW, RPW)
    ntab_p = jnp.pad(node_table, ((0, 0), (0, CW - C)))
    node_rows = _make_sc_gather()(nid, ntab_p)
    iflat = jnp.stack([idx[:, 0, :].reshape(BT), idx[:, 1, :].reshape(BT)])
    btcol = _bt_prep(iflat, time_table, day_table)
    deg3 = jnp.concatenate(
        [degrees, jnp.zeros((NBLK * CHUNK - N,), jnp.int32)]
    ).reshape(NBLK, 1, CHUNK)
    eye = jnp.eye(CHUNK, dtype=jnp.float32)
    out = _main(node_rows, deg3, degree_table, eye, y, btcol)
    return jnp.transpose(out.reshape(B, T, C, N), (0, 2, 3, 1))
